# baseline (speedup 1.0000x reference)
import functools

import jax
import jax.numpy as jnp
from jax import lax
from jax.experimental import pallas as pl
from jax.experimental.pallas import tpu as pltpu
from jax.experimental.pallas import tpu_sc as plsc

B = 1024
D = 4096
R = 5
NC = 2
NS = 16
L = 16
NW = NC * NS
RPW = B // NW
UN = 8

_mesh = plsc.VectorSubcoreMesh(core_axis_name="c", subcore_axis_name="s")


@functools.partial(
    pl.kernel,
    mesh=_mesh,
    out_type=(
        jax.ShapeDtypeStruct((B * R * D,), jnp.float32),
        jax.ShapeDtypeStruct((B,), jnp.float32),
    ),
    scratch_types=[
        pltpu.VMEM((RPW + L,), jnp.float32),
        pltpu.VMEM((RPW + L,), jnp.float32),
        pltpu.VMEM((D,), jnp.float32),
        pltpu.VMEM((4 * D,), jnp.float32),
        pltpu.VMEM((RPW,), jnp.float32),
    ],
)
def _sc_body(b_hbm, s_hbm, mats_hbm, ones_hbm, bvs, svs, vbuf, zbuf, obuf):
    wid = lax.axis_index("s") * NC + lax.axis_index("c")
    base = wid * RPW
    fi = lax.broadcasted_iota(jnp.int32, (L,), 0).astype(jnp.float32)
    zero = jnp.zeros((L,), jnp.float32)
    one = jnp.ones((L,), jnp.float32)

    pltpu.sync_copy(b_hbm.at[pl.ds(base, RPW)], bvs.at[pl.ds(0, RPW)])
    pltpu.sync_copy(s_hbm.at[pl.ds(base, RPW)], svs.at[pl.ds(0, RPW)])

    def zb(c, carry):
        zbuf[pl.ds(c * L, L)] = zero
        return carry

    lax.fori_loop(0, 4 * D // L, zb, 0)

    for g in range(RPW // L):
        obuf[pl.ds(g * L, L)] = one

    def row_body(r, carry):
        bb = jnp.full((L,), bvs[pl.ds(r, L)][0], jnp.float32)
        ss = jnp.full((L,), svs[pl.ds(r, L)][0], jnp.float32)
        v0 = bb + fi * ss
        deltas = [ss * jnp.float32(L * k) for k in range(UN)]
        stride = ss * jnp.float32(L * UN)

        def chunk(c, v):
            off = c * (L * UN)
            for k in range(UN):
                vbuf[pl.ds(off + k * L, L)] = v + deltas[k]
            return v + stride

        lax.fori_loop(0, D // (L * UN), chunk, v0)
        off0 = (base + r) * (R * D)
        pltpu.sync_copy(vbuf, mats_hbm.at[pl.ds(off0, D)])
        pltpu.sync_copy(zbuf, mats_hbm.at[pl.ds(off0 + D, 4 * D)])
        return carry

    lax.fori_loop(0, RPW, row_body, 0)
    pltpu.sync_copy(obuf, ones_hbm.at[pl.ds(base, RPW)])


def kernel(floats):
    b_arr = floats[:, 0]
    s_arr = (floats[:, 1] - floats[:, 0]) / jnp.float32(D - 1)
    mats_flat, ones_flat = _sc_body(b_arr, s_arr)
    return (mats_flat.reshape(B, R, D), ones_flat.reshape(B, 1))
